# baseline (speedup 1.0000x reference)
import functools

import jax
import jax.numpy as jnp
from jax import lax
from jax.experimental import pallas as pl
from jax.experimental.pallas import tpu as pltpu
from jax.experimental.pallas import tpu_sc as plsc

N = 10000
T = 16
D = 128
G3 = 3 * D
NP = 10240
NNZ = 160000
NNZP = 163840
PAD_ID = N

BN = 400


def _gru_body(x_ref, sl_ref, wih_ref, whh_ref, bih_ref, bhh_ref, h0_ref,
              w1t_ref, feat_ref, y1_ref):
    xr = x_ref[...]
    gi = jnp.dot(xr.reshape(BN * T, D).astype(jnp.bfloat16), wih_ref[...],
                 preferred_element_type=jnp.float32) + bih_ref[...]
    gi = gi.reshape(BN, T, G3)
    sl = sl_ref[...]
    idx = jnp.where(sl <= 0, T - 1, sl - 1)
    h = jnp.broadcast_to(h0_ref[...], (BN, D))
    feat = jnp.zeros((BN, D), jnp.float32)
    for t in range(T):
        gh = jnp.dot(h.astype(jnp.bfloat16), whh_ref[...],
                     preferred_element_type=jnp.float32) + bhh_ref[...]
        git = gi[:, t, :]
        r = jax.nn.sigmoid(git[:, 0:D] + gh[:, 0:D])
        z = jax.nn.sigmoid(git[:, D:2 * D] + gh[:, D:2 * D])
        n = jnp.tanh(git[:, 2 * D:] + r * gh[:, 2 * D:])
        h = (1.0 - z) * n + z * h
        feat = jnp.where(idx == t, h, feat)
    feat_ref[...] = feat
    y1_ref[...] = jnp.dot(feat.astype(jnp.bfloat16), w1t_ref[...],
                          preferred_element_type=jnp.float32)


def _run_gru(x, sl, wihT, whhT, bih, bhh, h0r, w1T):
    return pl.pallas_call(
        _gru_body,
        grid=(N // BN,),
        in_specs=[
            pl.BlockSpec((BN, T, D), lambda i: (i, 0, 0)),
            pl.BlockSpec((BN, 1), lambda i: (i, 0)),
            pl.BlockSpec((D, G3), lambda i: (0, 0)),
            pl.BlockSpec((D, G3), lambda i: (0, 0)),
            pl.BlockSpec((1, G3), lambda i: (0, 0)),
            pl.BlockSpec((1, G3), lambda i: (0, 0)),
            pl.BlockSpec((1, D), lambda i: (0, 0)),
            pl.BlockSpec((D, D), lambda i: (0, 0)),
        ],
        out_specs=[
            pl.BlockSpec((BN, D), lambda i: (i, 0)),
            pl.BlockSpec((BN, D), lambda i: (i, 0)),
        ],
        out_shape=[
            jax.ShapeDtypeStruct((N, D), jnp.float32),
            jax.ShapeDtypeStruct((N, D), jnp.float32),
        ],
    )(x, sl, wihT, whhT, bih, bhh, h0r, w1T)


CL = 128
CHUNKS = 40
HCH = 80
RPS = NP // 16


@functools.lru_cache(maxsize=None)
def _sc_apply_kernel():
    mesh = plsc.VectorSubcoreMesh(core_axis_name="c", subcore_axis_name="s")

    nbuf = 2
    ngrp = CHUNKS // nbuf

    @functools.partial(
        pl.kernel,
        out_type=jax.ShapeDtypeStruct((2, NP, D), jnp.float32),
        mesh=mesh,
        scratch_types=[
            pltpu.VMEM((CHUNKS, CL), jnp.int32),
            pltpu.VMEM((CHUNKS, CL), jnp.int32),
            pltpu.VMEM((nbuf * CL, D), jnp.float32),
            pltpu.VMEM_SHARED((NP, D), jnp.float32),
            pltpu.SemaphoreType.DMA,
        ],
    )
    def body_fn(src, gidx, sidx, zrows, out, gidx_v, sidx_v, rows_v, acc,
                gsem):
        cid = lax.axis_index("c")
        sid = lax.axis_index("s")
        pltpu.sync_copy(gidx.at[cid, sid], gidx_v)
        pltpu.sync_copy(sidx.at[cid, sid], sidx_v)
        pltpu.sync_copy(zrows, acc.at[pl.ds(sid * RPS, RPS)])
        plsc.subcore_barrier()

        def buf(b):
            return rows_v.at[pl.ds(b * CL, CL)]

        for b in range(nbuf):
            pltpu.async_copy(src.at[gidx_v.at[b]], buf(b), gsem)

        def group(g, carry):
            for b in range(nbuf):
                c = g * nbuf + b
                pltpu.make_async_copy(src.at[gidx_v.at[c]], buf(b),
                                      gsem).wait()
                pltpu.sync_copy(buf(b), acc.at[sidx_v.at[c]], add=True)

                @pl.when(g < ngrp - 1)
                def _():
                    pltpu.async_copy(src.at[gidx_v.at[c + nbuf]], buf(b),
                                     gsem)
            return carry

        lax.fori_loop(0, ngrp, group, 0)
        plsc.subcore_barrier()
        pltpu.sync_copy(acc.at[pl.ds(sid * RPS, RPS)],
                        out.at[cid, pl.ds(sid * RPS, RPS)])

    return body_fn


def _sc_apply(src, gidx, sidx, zrows):
    return _sc_apply_kernel()(src, gidx, sidx, zrows)


@functools.lru_cache(maxsize=None)
def _sc_hist_kernel():
    mesh = plsc.VectorSubcoreMesh(core_axis_name="c", subcore_axis_name="s")

    @functools.partial(
        pl.kernel,
        out_type=jax.ShapeDtypeStruct((2, NP), jnp.float32),
        mesh=mesh,
        scratch_types=[
            pltpu.VMEM((HCH, CL), jnp.int32),
            pltpu.VMEM((CL,), jnp.float32),
            pltpu.VMEM_SHARED((NP,), jnp.float32),
        ],
    )
    def body_fn(idx2, zvec, ones, out, idx_v, ones_v, acc):
        cid = lax.axis_index("c")
        sid = lax.axis_index("s")
        pltpu.sync_copy(idx2.at[cid, sid], idx_v)
        pltpu.sync_copy(ones, ones_v)
        pltpu.sync_copy(zvec, acc.at[pl.ds(sid * RPS, RPS)])
        plsc.subcore_barrier()

        def body(c, carry):
            pltpu.sync_copy(ones_v, acc.at[idx_v.at[c]], add=True)
            return carry

        lax.fori_loop(0, HCH, body, 0)
        plsc.subcore_barrier()
        pltpu.sync_copy(acc.at[pl.ds(sid * RPS, RPS)],
                        out.at[cid, pl.ds(sid * RPS, RPS)])

    return body_fn


def _sc_hist(idx2, zvec, ones):
    return _sc_hist_kernel()(idx2, zvec, ones)


BM = 512


def _scale_body(m0_ref, m1_ref, cnt_ref, out_ref):
    cnt = cnt_ref[...]
    inv = jnp.where(cnt > 0, 1.0 / cnt, 0.0)
    out_ref[...] = (m0_ref[...] + m1_ref[...]) * inv


def _run_scale(m0, m1, cnt):
    return pl.pallas_call(
        _scale_body,
        grid=(NP // BM,),
        in_specs=[
            pl.BlockSpec((BM, D), lambda i: (i, 0)),
            pl.BlockSpec((BM, D), lambda i: (i, 0)),
            pl.BlockSpec((BM, 1), lambda i: (i, 0)),
        ],
        out_specs=pl.BlockSpec((BM, D), lambda i: (i, 0)),
        out_shape=jax.ShapeDtypeStruct((NP, D), jnp.float32),
    )(m0, m1, cnt)


def _make_comb_body(mask_rows):
    def body(s0_ref, s1_ref, cnt_ref, b_ref, prev_ref, wt_ref, o_ref, y_ref):
        cnt = cnt_ref[...]
        inv = jnp.where(cnt > 0, 1.0 / cnt, 0.0)
        o = (s0_ref[...] + s1_ref[...]) * inv + b_ref[...] + prev_ref[...]
        if mask_rows:
            row = (lax.broadcasted_iota(jnp.int32, (BM, 1), 0)
                   + pl.program_id(0) * BM)
            o = jnp.where(row < N, o, 0.0)
        o_ref[...] = o
        y_ref[...] = jnp.dot(o, wt_ref[...], preferred_element_type=jnp.float32)
    return body


def _run_comb(s0, s1, cnt, b, prev, wt, mask_rows):
    return pl.pallas_call(
        _make_comb_body(mask_rows),
        grid=(NP // BM,),
        in_specs=[
            pl.BlockSpec((BM, D), lambda i: (i, 0)),
            pl.BlockSpec((BM, D), lambda i: (i, 0)),
            pl.BlockSpec((BM, 1), lambda i: (i, 0)),
            pl.BlockSpec((1, D), lambda i: (0, 0)),
            pl.BlockSpec((BM, D), lambda i: (i, 0)),
            pl.BlockSpec((D, D), lambda i: (0, 0)),
        ],
        out_specs=[
            pl.BlockSpec((BM, D), lambda i: (i, 0)),
            pl.BlockSpec((BM, D), lambda i: (i, 0)),
        ],
        out_shape=[
            jax.ShapeDtypeStruct((NP, D), jnp.float32),
            jax.ShapeDtypeStruct((NP, D), jnp.float32),
        ],
    )(s0, s1, cnt, b, prev, wt)


BA = 512
NB = NP // BA
_INV_DD = 1.0 / float(D * D)


def _deg_body(o3_ref, o3t_ref, phi_ref, dis_ref):
    i = pl.program_id(0)
    j = pl.program_id(1)
    a = jnp.dot(o3_ref[...].astype(jnp.bfloat16),
                o3t_ref[...].astype(jnp.bfloat16),
                preferred_element_type=jnp.float32) * _INV_DD
    hit = (a >= phi_ref[0, 0]).astype(jnp.float32)
    ri = lax.broadcasted_iota(jnp.int32, (BA, BA), 0) + i * BA
    ci = lax.broadcasted_iota(jnp.int32, (BA, BA), 1) + j * BA
    selfhit = jnp.sum(jnp.where(ri == ci, hit, 0.0), axis=1, keepdims=True)
    cnt = jnp.sum(hit, axis=1, keepdims=True) - selfhit

    @pl.when(j == 0)
    def _():
        dis_ref[...] = cnt + 1.0

    @pl.when(j > 0)
    def _():
        dis_ref[...] += cnt

    @pl.when(j == NB - 1)
    def _():
        dis_ref[...] = lax.rsqrt(dis_ref[...])


def _run_deg(o3, o3t, phi):
    return pl.pallas_call(
        _deg_body,
        grid=(NB, NB),
        in_specs=[
            pl.BlockSpec((BA, D), lambda i, j: (i, 0)),
            pl.BlockSpec((D, BA), lambda i, j: (0, j)),
            pl.BlockSpec((1, 1), lambda i, j: (0, 0)),
        ],
        out_specs=pl.BlockSpec((BA, 1), lambda i, j: (i, 0)),
        out_shape=jax.ShapeDtypeStruct((NP, 1), jnp.float32),
    )(o3, o3t, phi)


def _gcn_body(o3_ref, o3t_ref, xg_ref, disi_ref, disj_ref, phi_ref, bg_ref,
              out_ref):
    i = pl.program_id(0)
    j = pl.program_id(1)
    a = jnp.dot(o3_ref[...].astype(jnp.bfloat16),
                o3t_ref[...].astype(jnp.bfloat16),
                preferred_element_type=jnp.float32) * _INV_DD
    hit = (a >= phi_ref[0, 0]).astype(jnp.float32)
    ri = lax.broadcasted_iota(jnp.int32, (BA, BA), 0) + i * BA
    ci = lax.broadcasted_iota(jnp.int32, (BA, BA), 1) + j * BA
    adj = jnp.where(ri == ci, 1.0, hit)
    u = disj_ref[...] * xg_ref[...]
    part = jnp.dot(adj.astype(jnp.bfloat16), u.astype(jnp.bfloat16),
                   preferred_element_type=jnp.float32)

    @pl.when(j == 0)
    def _():
        out_ref[...] = part

    @pl.when(j > 0)
    def _():
        out_ref[...] += part

    @pl.when(j == NB - 1)
    def _():
        out_ref[...] = disi_ref[...] * out_ref[...] + bg_ref[...]


def _run_gcn(o3, o3t, xg, dis, phi, bg):
    return pl.pallas_call(
        _gcn_body,
        grid=(NB, NB),
        in_specs=[
            pl.BlockSpec((BA, D), lambda i, j: (i, 0)),
            pl.BlockSpec((D, BA), lambda i, j: (0, j)),
            pl.BlockSpec((BA, D), lambda i, j: (j, 0)),
            pl.BlockSpec((BA, 1), lambda i, j: (i, 0)),
            pl.BlockSpec((BA, 1), lambda i, j: (j, 0)),
            pl.BlockSpec((1, 1), lambda i, j: (0, 0)),
            pl.BlockSpec((1, D), lambda i, j: (0, 0)),
        ],
        out_specs=pl.BlockSpec((BA, D), lambda i, j: (i, 0)),
        out_shape=jax.ShapeDtypeStruct((NP, D), jnp.float32),
    )(o3, o3t, xg, dis, dis, phi, bg)


def kernel(x, hyperedge_index, sorted_length, W_ih, W_hh, b_ih, b_hh, h0,
           W1, b1, W2, b2, W3, b3, phi, Wg, bg):
    f32 = jnp.float32
    bf16 = jnp.bfloat16
    sl = sorted_length.astype(jnp.int32).reshape(N, 1)
    wihT = W_ih.T.astype(bf16)
    whhT = W_hh.T.astype(bf16)
    bih = b_ih.reshape(1, G3)
    bhh = b_hh.reshape(1, G3)
    h0r = h0.reshape(1, D)

    feat, y1 = _run_gru(x, sl, wihT, whhT, bih, bhh, h0r, W1.T.astype(bf16))
    featP = jnp.pad(feat, ((0, NP - N), (0, 0)))
    y1P = jnp.pad(y1, ((0, NP - N), (0, 0)))

    hi = jnp.concatenate(
        [hyperedge_index.astype(jnp.int32),
         jnp.full((2, NNZP - NNZ), PAD_ID, jnp.int32)], axis=1)
    conv_node = hi[0].reshape(2, 16, CHUNKS, CL)
    conv_edge = hi[1].reshape(2, 16, CHUNKS, CL)
    hist_idx = hi.reshape(2, 16, HCH, CL)
    zrows = jnp.zeros((RPS, D), f32)
    zvec = jnp.zeros((RPS,), f32)
    ones = jnp.ones((CL,), f32)

    cnts = _sc_hist(hist_idx, zvec, ones)
    Dn = cnts[0].reshape(NP, 1)
    Bn = cnts[1].reshape(NP, 1)

    def conv(Y, prev, b, wnextT, mask_rows):
        m = _sc_apply(Y, conv_node, conv_edge, zrows)
        Z = _run_scale(m[0], m[1], Bn)
        s = _sc_apply(Z, conv_edge, conv_node, zrows)
        return _run_comb(s[0], s[1], Dn, b.reshape(1, D), prev, wnextT,
                         mask_rows)

    o1, y2 = conv(y1P, featP, b1, W2.T, False)
    o2, y3 = conv(y2, o1, b2, W3.T, False)
    o3, xg = conv(y3, o2, b3, Wg.T, True)

    phiR = jnp.reshape(phi, (1, 1)).astype(f32)
    o3t = o3.T
    dis = _run_deg(o3, o3t, phiR)
    out = _run_gcn(o3, o3t, xg, dis, phiR, bg.reshape(1, D))
    return out[:N]


# speedup vs baseline: 1.0658x; 1.0658x over previous
import functools

import jax
import jax.numpy as jnp
from jax import lax
from jax.experimental import pallas as pl
from jax.experimental.pallas import tpu as pltpu
from jax.experimental.pallas import tpu_sc as plsc

N = 10000
T = 16
D = 128
G3 = 3 * D
NP = 10240
NNZ = 160000
NNZP = 163840
PAD_ID = N

BN = 400


def _gru_body(x_ref, sl_ref, wih_ref, whh_ref, bih_ref, bhh_ref, h0_ref,
              w1t_ref, feat_ref, y1_ref):
    xr = x_ref[...]
    gi = jnp.dot(xr.reshape(BN * T, D), wih_ref[...],
                 preferred_element_type=jnp.float32) + bih_ref[...]
    gi = gi.reshape(BN, T, G3)
    sl = sl_ref[...]
    idx = jnp.where(sl <= 0, T - 1, sl - 1)
    h = jnp.broadcast_to(h0_ref[...], (BN, D))
    feat = jnp.zeros((BN, D), jnp.float32)
    for t in range(T):
        gh = jnp.dot(h, whh_ref[...],
                     preferred_element_type=jnp.float32) + bhh_ref[...]
        git = gi[:, t, :]
        r = jax.nn.sigmoid(git[:, 0:D] + gh[:, 0:D])
        z = jax.nn.sigmoid(git[:, D:2 * D] + gh[:, D:2 * D])
        n = jnp.tanh(git[:, 2 * D:] + r * gh[:, 2 * D:])
        h = (1.0 - z) * n + z * h
        feat = jnp.where(idx == t, h, feat)
    feat_ref[...] = feat
    y1_ref[...] = jnp.dot(feat, w1t_ref[...],
                          preferred_element_type=jnp.float32)


def _run_gru(x, sl, wihT, whhT, bih, bhh, h0r, w1T):
    return pl.pallas_call(
        _gru_body,
        grid=(N // BN,),
        in_specs=[
            pl.BlockSpec((BN, T, D), lambda i: (i, 0, 0)),
            pl.BlockSpec((BN, 1), lambda i: (i, 0)),
            pl.BlockSpec((D, G3), lambda i: (0, 0)),
            pl.BlockSpec((D, G3), lambda i: (0, 0)),
            pl.BlockSpec((1, G3), lambda i: (0, 0)),
            pl.BlockSpec((1, G3), lambda i: (0, 0)),
            pl.BlockSpec((1, D), lambda i: (0, 0)),
            pl.BlockSpec((D, D), lambda i: (0, 0)),
        ],
        out_specs=[
            pl.BlockSpec((BN, D), lambda i: (i, 0)),
            pl.BlockSpec((BN, D), lambda i: (i, 0)),
        ],
        out_shape=[
            jax.ShapeDtypeStruct((N, D), jnp.float32),
            jax.ShapeDtypeStruct((N, D), jnp.float32),
        ],
    )(x, sl, wihT, whhT, bih, bhh, h0r, w1T)


CL = 128
CHUNKS = 40
HCH = 80
RPS = NP // 16


@functools.lru_cache(maxsize=None)
def _sc_apply_kernel():
    mesh = plsc.VectorSubcoreMesh(core_axis_name="c", subcore_axis_name="s")

    nbuf = 2
    ngrp = CHUNKS // nbuf

    @functools.partial(
        pl.kernel,
        out_type=jax.ShapeDtypeStruct((2, NP, D), jnp.float32),
        mesh=mesh,
        scratch_types=[
            pltpu.VMEM((CHUNKS, CL), jnp.int32),
            pltpu.VMEM((CHUNKS, CL), jnp.int32),
            pltpu.VMEM((nbuf * CL, D), jnp.float32),
            pltpu.VMEM_SHARED((NP, D), jnp.float32),
            pltpu.SemaphoreType.DMA,
        ],
    )
    def body_fn(src, gidx, sidx, zrows, out, gidx_v, sidx_v, rows_v, acc,
                gsem):
        cid = lax.axis_index("c")
        sid = lax.axis_index("s")
        pltpu.sync_copy(gidx.at[cid, sid], gidx_v)
        pltpu.sync_copy(sidx.at[cid, sid], sidx_v)
        pltpu.sync_copy(zrows, acc.at[pl.ds(sid * RPS, RPS)])
        plsc.subcore_barrier()

        def buf(b):
            return rows_v.at[pl.ds(b * CL, CL)]

        for b in range(nbuf):
            pltpu.async_copy(src.at[gidx_v.at[b]], buf(b), gsem)

        def group(g, carry):
            for b in range(nbuf):
                c = g * nbuf + b
                pltpu.make_async_copy(src.at[gidx_v.at[c]], buf(b),
                                      gsem).wait()
                pltpu.sync_copy(buf(b), acc.at[sidx_v.at[c]], add=True)

                @pl.when(g < ngrp - 1)
                def _():
                    pltpu.async_copy(src.at[gidx_v.at[c + nbuf]], buf(b),
                                     gsem)
            return carry

        lax.fori_loop(0, ngrp, group, 0)
        plsc.subcore_barrier()
        pltpu.sync_copy(acc.at[pl.ds(sid * RPS, RPS)],
                        out.at[cid, pl.ds(sid * RPS, RPS)])

    return body_fn


def _sc_apply(src, gidx, sidx, zrows):
    return _sc_apply_kernel()(src, gidx, sidx, zrows)


@functools.lru_cache(maxsize=None)
def _sc_hist_kernel():
    mesh = plsc.VectorSubcoreMesh(core_axis_name="c", subcore_axis_name="s")

    @functools.partial(
        pl.kernel,
        out_type=jax.ShapeDtypeStruct((2, NP), jnp.float32),
        mesh=mesh,
        scratch_types=[
            pltpu.VMEM((HCH, CL), jnp.int32),
            pltpu.VMEM((CL,), jnp.float32),
            pltpu.VMEM_SHARED((NP,), jnp.float32),
        ],
    )
    def body_fn(idx2, zvec, ones, out, idx_v, ones_v, acc):
        cid = lax.axis_index("c")
        sid = lax.axis_index("s")
        pltpu.sync_copy(idx2.at[cid, sid], idx_v)
        pltpu.sync_copy(ones, ones_v)
        pltpu.sync_copy(zvec, acc.at[pl.ds(sid * RPS, RPS)])
        plsc.subcore_barrier()

        def body(c, carry):
            pltpu.sync_copy(ones_v, acc.at[idx_v.at[c]], add=True)
            return carry

        lax.fori_loop(0, HCH, body, 0)
        plsc.subcore_barrier()
        pltpu.sync_copy(acc.at[pl.ds(sid * RPS, RPS)],
                        out.at[cid, pl.ds(sid * RPS, RPS)])

    return body_fn


def _sc_hist(idx2, zvec, ones):
    return _sc_hist_kernel()(idx2, zvec, ones)


BM = 512


def _scale_body(m0_ref, m1_ref, cnt_ref, out_ref):
    cnt = cnt_ref[...]
    inv = jnp.where(cnt > 0, 1.0 / cnt, 0.0)
    out_ref[...] = (m0_ref[...] + m1_ref[...]) * inv


def _run_scale(m0, m1, cnt):
    return pl.pallas_call(
        _scale_body,
        grid=(NP // BM,),
        in_specs=[
            pl.BlockSpec((BM, D), lambda i: (i, 0)),
            pl.BlockSpec((BM, D), lambda i: (i, 0)),
            pl.BlockSpec((BM, 1), lambda i: (i, 0)),
        ],
        out_specs=pl.BlockSpec((BM, D), lambda i: (i, 0)),
        out_shape=jax.ShapeDtypeStruct((NP, D), jnp.float32),
    )(m0, m1, cnt)


def _make_comb_body(mask_rows):
    def body(s0_ref, s1_ref, cnt_ref, b_ref, prev_ref, wt_ref, o_ref, y_ref):
        cnt = cnt_ref[...]
        inv = jnp.where(cnt > 0, 1.0 / cnt, 0.0)
        o = (s0_ref[...] + s1_ref[...]) * inv + b_ref[...] + prev_ref[...]
        if mask_rows:
            row = (lax.broadcasted_iota(jnp.int32, (BM, 1), 0)
                   + pl.program_id(0) * BM)
            o = jnp.where(row < N, o, 0.0)
        o_ref[...] = o
        y_ref[...] = jnp.dot(o, wt_ref[...], preferred_element_type=jnp.float32)
    return body


def _run_comb(s0, s1, cnt, b, prev, wt, mask_rows):
    return pl.pallas_call(
        _make_comb_body(mask_rows),
        grid=(NP // BM,),
        in_specs=[
            pl.BlockSpec((BM, D), lambda i: (i, 0)),
            pl.BlockSpec((BM, D), lambda i: (i, 0)),
            pl.BlockSpec((BM, 1), lambda i: (i, 0)),
            pl.BlockSpec((1, D), lambda i: (0, 0)),
            pl.BlockSpec((BM, D), lambda i: (i, 0)),
            pl.BlockSpec((D, D), lambda i: (0, 0)),
        ],
        out_specs=[
            pl.BlockSpec((BM, D), lambda i: (i, 0)),
            pl.BlockSpec((BM, D), lambda i: (i, 0)),
        ],
        out_shape=[
            jax.ShapeDtypeStruct((NP, D), jnp.float32),
            jax.ShapeDtypeStruct((NP, D), jnp.float32),
        ],
    )(s0, s1, cnt, b, prev, wt)


BA = 512
NB = NP // BA
_INV_DD = 1.0 / float(D * D)


def _deg_body(o3_ref, o3t_ref, phi_ref, dis_ref):
    i = pl.program_id(0)
    j = pl.program_id(1)
    a = jnp.dot(o3_ref[...], o3t_ref[...],
                preferred_element_type=jnp.float32) * _INV_DD
    hit = (a >= phi_ref[0, 0]).astype(jnp.float32)
    cnt = jnp.sum(hit, axis=1, keepdims=True)

    @pl.when(j == 0)
    def _():
        dis_ref[...] = cnt + 1.0

    @pl.when(j > 0)
    def _():
        dis_ref[...] += cnt

    @pl.when(j == i)
    def _():
        ri = lax.broadcasted_iota(jnp.int32, (BA, BA), 0)
        ci = lax.broadcasted_iota(jnp.int32, (BA, BA), 1)
        selfhit = jnp.sum(jnp.where(ri == ci, hit, 0.0), axis=1,
                          keepdims=True)
        dis_ref[...] -= selfhit

    @pl.when(j == NB - 1)
    def _():
        dis_ref[...] = lax.rsqrt(dis_ref[...])


def _run_deg(o3, o3t, phi):
    return pl.pallas_call(
        _deg_body,
        grid=(NB, NB),
        in_specs=[
            pl.BlockSpec((BA, D), lambda i, j: (i, 0)),
            pl.BlockSpec((D, BA), lambda i, j: (0, j)),
            pl.BlockSpec((1, 1), lambda i, j: (0, 0)),
        ],
        out_specs=pl.BlockSpec((BA, 1), lambda i, j: (i, 0)),
        out_shape=jax.ShapeDtypeStruct((NP, 1), jnp.float32),
    )(o3, o3t, phi)


def _gcn_body(o3_ref, o3t_ref, xg_ref, disi_ref, disj_ref, phi_ref, bg_ref,
              out_ref):
    i = pl.program_id(0)
    j = pl.program_id(1)
    a = jnp.dot(o3_ref[...], o3t_ref[...],
                preferred_element_type=jnp.float32) * _INV_DD
    hit = (a >= phi_ref[0, 0]).astype(jnp.float32)
    u = disj_ref[...] * xg_ref[...]
    part = jnp.dot(hit, u, preferred_element_type=jnp.float32)

    @pl.when(j == 0)
    def _():
        out_ref[...] = part

    @pl.when(j > 0)
    def _():
        out_ref[...] += part

    @pl.when(j == i)
    def _():
        ri = lax.broadcasted_iota(jnp.int32, (BA, BA), 0)
        ci = lax.broadcasted_iota(jnp.int32, (BA, BA), 1)
        dhit = jnp.sum(jnp.where(ri == ci, hit, 0.0), axis=1, keepdims=True)
        out_ref[...] += (1.0 - dhit) * u

    @pl.when(j == NB - 1)
    def _():
        out_ref[...] = disi_ref[...] * out_ref[...] + bg_ref[...]


def _run_gcn(o3, o3t, xg, dis, phi, bg):
    return pl.pallas_call(
        _gcn_body,
        grid=(NB, NB),
        in_specs=[
            pl.BlockSpec((BA, D), lambda i, j: (i, 0)),
            pl.BlockSpec((D, BA), lambda i, j: (0, j)),
            pl.BlockSpec((BA, D), lambda i, j: (j, 0)),
            pl.BlockSpec((BA, 1), lambda i, j: (i, 0)),
            pl.BlockSpec((BA, 1), lambda i, j: (j, 0)),
            pl.BlockSpec((1, 1), lambda i, j: (0, 0)),
            pl.BlockSpec((1, D), lambda i, j: (0, 0)),
        ],
        out_specs=pl.BlockSpec((BA, D), lambda i, j: (i, 0)),
        out_shape=jax.ShapeDtypeStruct((NP, D), jnp.float32),
    )(o3, o3t, xg, dis, dis, phi, bg)


def kernel(x, hyperedge_index, sorted_length, W_ih, W_hh, b_ih, b_hh, h0,
           W1, b1, W2, b2, W3, b3, phi, Wg, bg):
    f32 = jnp.float32
    sl = sorted_length.astype(jnp.int32).reshape(N, 1)
    wihT = W_ih.T
    whhT = W_hh.T
    bih = b_ih.reshape(1, G3)
    bhh = b_hh.reshape(1, G3)
    h0r = h0.reshape(1, D)

    feat, y1 = _run_gru(x, sl, wihT, whhT, bih, bhh, h0r, W1.T)
    featP = jnp.pad(feat, ((0, NP - N), (0, 0)))
    y1P = jnp.pad(y1, ((0, NP - N), (0, 0)))

    hi = jnp.concatenate(
        [hyperedge_index.astype(jnp.int32),
         jnp.full((2, NNZP - NNZ), PAD_ID, jnp.int32)], axis=1)
    conv_node = hi[0].reshape(2, 16, CHUNKS, CL)
    conv_edge = hi[1].reshape(2, 16, CHUNKS, CL)
    hist_idx = hi.reshape(2, 16, HCH, CL)
    zrows = jnp.zeros((RPS, D), f32)
    zvec = jnp.zeros((RPS,), f32)
    ones = jnp.ones((CL,), f32)

    cnts = _sc_hist(hist_idx, zvec, ones)
    Dn = cnts[0].reshape(NP, 1)
    Bn = cnts[1].reshape(NP, 1)

    def conv(Y, prev, b, wnextT, mask_rows):
        m = _sc_apply(Y, conv_node, conv_edge, zrows)
        Z = _run_scale(m[0], m[1], Bn)
        s = _sc_apply(Z, conv_edge, conv_node, zrows)
        return _run_comb(s[0], s[1], Dn, b.reshape(1, D), prev, wnextT,
                         mask_rows)

    o1, y2 = conv(y1P, featP, b1, W2.T, False)
    o2, y3 = conv(y2, o1, b2, W3.T, False)
    o3, xg = conv(y3, o2, b3, Wg.T, True)

    phiR = jnp.reshape(phi, (1, 1)).astype(f32)
    o3t = o3.T
    dis = _run_deg(o3, o3t, phiR)
    out = _run_gcn(o3, o3t, xg, dis, phiR, bg.reshape(1, D))
    return out[:N]
